# baseline (speedup 1.0000x reference)
import functools

import jax
import jax.numpy as jnp
from jax import lax
from jax.experimental import pallas as pl
from jax.experimental.pallas import tpu as pltpu
from jax.experimental.pallas import tpu_sc as plsc

N = 10000
E = 320000
F_IN = 128
H = 128
C = 2

NC = 2
NS = 16
NW = NC * NS
EPT = E // NW
CH = 80
NCH = EPT // CH
NPAD = 10240
STRIPE = NPAD // NS

_mesh = plsc.VectorSubcoreMesh(
    core_axis_name="c", subcore_axis_name="s", num_cores=NC, num_subcores=NS
)


@functools.partial(
    pl.kernel,
    out_type=(
        jax.ShapeDtypeStruct((NC * NPAD,), jnp.float32),
        jax.ShapeDtypeStruct((NC * NPAD,), jnp.float32),
    ),
    mesh=_mesh,
    scratch_types=[
        pltpu.VMEM((NCH, CH), jnp.int32),
        pltpu.VMEM((NCH, CH), jnp.int32),
        pltpu.VMEM((NCH, CH), jnp.float32),
        pltpu.VMEM((CH,), jnp.float32),
        pltpu.VMEM((STRIPE,), jnp.float32),
        pltpu.VMEM_SHARED((NPAD,), jnp.float32),
        pltpu.VMEM_SHARED((NPAD,), jnp.float32),
    ],
)
def _deg_kernel(src_hbm, dst_hbm, deg_out, sl_out,
                sidx, didx, slv, ones, stage, deg_s, sl_s):
    cid = lax.axis_index("c")
    sid = lax.axis_index("s")
    wid = sid * NC + cid

    one_v = jnp.ones((16,), jnp.float32)
    zero_v = jnp.zeros((16,), jnp.float32)
    for c in range(CH // 16):
        ones[pl.ds(c * 16, 16)] = one_v

    def zfill(i, _):
        stage[pl.ds(i * 16, 16)] = zero_v
        return 0
    lax.fori_loop(0, STRIPE // 16, zfill, 0)

    base = sid * STRIPE
    pltpu.sync_copy(stage, deg_s.at[pl.ds(base, STRIPE)])
    pltpu.sync_copy(stage, sl_s.at[pl.ds(base, STRIPE)])
    plsc.subcore_barrier()

    pltpu.sync_copy(src_hbm.at[wid], sidx)
    pltpu.sync_copy(dst_hbm.at[wid], didx)

    def slvals(j, _):
        for c in range(CH // 16):
            s = sidx[j, pl.ds(c * 16, 16)]
            d = didx[j, pl.ds(c * 16, 16)]
            slv[j, pl.ds(c * 16, 16)] = jnp.where(s == d, 1.0, 0.0).astype(
                jnp.float32)
        return 0
    lax.fori_loop(0, NCH, slvals, 0)

    def accum(j, _):
        pltpu.sync_copy(ones, deg_s.at[sidx.at[j]], add=True)
        pltpu.sync_copy(slv.at[j], sl_s.at[sidx.at[j]], add=True)
        return 0
    lax.fori_loop(0, NCH, accum, 0)
    plsc.subcore_barrier()

    pltpu.sync_copy(deg_s.at[pl.ds(base, STRIPE)], stage)
    pltpu.sync_copy(stage, deg_out.at[pl.ds(cid * NPAD + base, STRIPE)])
    pltpu.sync_copy(sl_s.at[pl.ds(base, STRIPE)], stage)
    pltpu.sync_copy(stage, sl_out.at[pl.ds(cid * NPAD + base, STRIPE)])


SCH = 80
IBLK = 25
NBLK = EPT // (IBLK * SCH)
ZCH = 80


@functools.partial(
    pl.kernel,
    out_type=jax.ShapeDtypeStruct((NC, NPAD, H), jnp.float32),
    mesh=_mesh,
    scratch_types=[
        pltpu.VMEM((IBLK, SCH), jnp.int32),
        pltpu.VMEM((IBLK, SCH), jnp.int32),
        pltpu.VMEM((SCH, H), jnp.float32),
        pltpu.VMEM((SCH, H), jnp.float32),
        pltpu.VMEM_SHARED((NPAD, H), jnp.float32),
        pltpu.SemaphoreType.DMA,
        pltpu.SemaphoreType.DMA,
        pltpu.SemaphoreType.DMA,
        pltpu.SemaphoreType.DMA,
    ],
)
def _seg_kernel(y_hbm, src_hbm, dst_hbm, s_out, sidx, didx, rows0, rows1, S,
                sem0, sem1, ssem0, ssem1):
    cid = lax.axis_index("c")
    sid = lax.axis_index("s")
    wid = sid * NC + cid

    zero_v = jnp.zeros((16,), jnp.float32)

    def zrow(r, _):
        for c in range(H // 16):
            rows0[r, pl.ds(c * 16, 16)] = zero_v
        return 0
    lax.fori_loop(0, ZCH, zrow, 0)

    base = sid * STRIPE
    zsrc = rows0.at[pl.ds(0, ZCH), :]

    def zstripe(i, _):
        pltpu.sync_copy(zsrc, S.at[pl.ds(base + i * ZCH, ZCH), :])
        return 0
    lax.fori_loop(0, STRIPE // ZCH, zstripe, 0)
    plsc.subcore_barrier()

    def blk(b, _):
        pltpu.sync_copy(src_hbm.at[wid, b], sidx)
        pltpu.sync_copy(dst_hbm.at[wid, b], didx)
        pltpu.async_copy(y_hbm.at[sidx.at[0]], rows0, sem0)
        pltpu.async_copy(y_hbm.at[sidx.at[1]], rows1, sem1)

        def pair(i, _):
            c0 = 2 * i
            c1 = c0 + 1
            pltpu.make_async_copy(y_hbm.at[sidx.at[c0]], rows0, sem0).wait()
            pltpu.async_copy(rows0, S.at[didx.at[c0]], ssem0, add=True)
            pltpu.make_async_copy(y_hbm.at[sidx.at[c1]], rows1, sem1).wait()
            pltpu.async_copy(rows1, S.at[didx.at[c1]], ssem1, add=True)
            pltpu.make_async_copy(rows0, S.at[didx.at[c0]], ssem0).wait()
            pltpu.async_copy(y_hbm.at[sidx.at[c0 + 2]], rows0, sem0)

            @pl.when(c1 + 2 < IBLK)
            def _():
                pltpu.make_async_copy(rows1, S.at[didx.at[c1]], ssem1).wait()
                pltpu.async_copy(y_hbm.at[sidx.at[c1 + 2]], rows1, sem1)
            return 0
        lax.fori_loop(0, IBLK // 2, pair, 0)
        last = IBLK - 1
        pltpu.make_async_copy(rows1, S.at[didx.at[last]], ssem1).wait()
        pltpu.make_async_copy(y_hbm.at[sidx.at[last]], rows0, sem0).wait()
        pltpu.sync_copy(rows0, S.at[didx.at[last]], add=True)
        return 0
    lax.fori_loop(0, NBLK, blk, 0)
    plsc.subcore_barrier()

    def rd(i, _):
        pltpu.sync_copy(S.at[pl.ds(base + i * ZCH, ZCH), :],
                        s_out.at[cid, pl.ds(base + i * ZCH, ZCH), :])
        return 0
    lax.fori_loop(0, STRIPE // ZCH, rd, 0)


BLK = 1280
GRID = NPAD // BLK


def _prep_body(degp_ref, slp_ref, x_ref, dinv_ref, slc_ref, y_ref):
    deg_full = degp_ref[0, :] + degp_ref[1, :]
    slc = slp_ref[0, :] + slp_ref[1, :]
    deg = deg_full - slc
    dinv = jnp.where(deg > 0, lax.rsqrt(jnp.maximum(deg, 1e-30)), 0.0)
    dinv_ref[...] = dinv[:, None]
    slc_ref[...] = slc[:, None]
    y_ref[...] = dinv[:, None] * x_ref[...]


def _prep(degp, slp, xp):
    return pl.pallas_call(
        _prep_body,
        grid=(GRID,),
        in_specs=[
            pl.BlockSpec((NC, BLK), lambda i: (0, i)),
            pl.BlockSpec((NC, BLK), lambda i: (0, i)),
            pl.BlockSpec((BLK, H), lambda i: (i, 0)),
        ],
        out_specs=[
            pl.BlockSpec((BLK, 1), lambda i: (i, 0)),
            pl.BlockSpec((BLK, 1), lambda i: (i, 0)),
            pl.BlockSpec((BLK, H), lambda i: (i, 0)),
        ],
        out_shape=[
            jax.ShapeDtypeStruct((NPAD, 1), jnp.float32),
            jax.ShapeDtypeStruct((NPAD, 1), jnp.float32),
            jax.ShapeDtypeStruct((NPAD, H), jnp.float32),
        ],
    )(degp, slp, xp)


def _layer_body(x_ref, y_ref, s_ref, dinv_ref, slc_ref, w0_ref, w1_ref, b_ref,
                h_ref, y2_ref):
    dinv = dinv_ref[...]
    tx1 = -dinv * (s_ref[0] + s_ref[1] - slc_ref[...] * y_ref[...])
    h = (jnp.dot(x_ref[...], w0_ref[...], preferred_element_type=jnp.float32)
         + jnp.dot(tx1, w1_ref[...], preferred_element_type=jnp.float32)
         + b_ref[...])
    h = jnp.maximum(h, 0.0)
    h_ref[...] = h
    y2_ref[...] = dinv * h


def _layer(xp, y, s, dinv, slc, w0, w1, b):
    return pl.pallas_call(
        _layer_body,
        grid=(GRID,),
        in_specs=[
            pl.BlockSpec((BLK, H), lambda i: (i, 0)),
            pl.BlockSpec((BLK, H), lambda i: (i, 0)),
            pl.BlockSpec((NC, BLK, H), lambda i: (0, i, 0)),
            pl.BlockSpec((BLK, 1), lambda i: (i, 0)),
            pl.BlockSpec((BLK, 1), lambda i: (i, 0)),
            pl.BlockSpec((H, H), lambda i: (0, 0)),
            pl.BlockSpec((H, H), lambda i: (0, 0)),
            pl.BlockSpec((1, H), lambda i: (0, 0)),
        ],
        out_specs=[
            pl.BlockSpec((BLK, H), lambda i: (i, 0)),
            pl.BlockSpec((BLK, H), lambda i: (i, 0)),
        ],
        out_shape=[
            jax.ShapeDtypeStruct((NPAD, H), jnp.float32),
            jax.ShapeDtypeStruct((NPAD, H), jnp.float32),
        ],
    )(xp, y, s, dinv, slc, w0, w1, b)


def _final_body(h_ref, y_ref, s_ref, dinv_ref, slc_ref, w0_ref, w1_ref, b_ref,
                wl_ref, bl_ref, out_ref):
    dinv = dinv_ref[...]
    tx1 = -dinv * (s_ref[0] + s_ref[1] - slc_ref[...] * y_ref[...])
    h = (jnp.dot(h_ref[...], w0_ref[...], preferred_element_type=jnp.float32)
         + jnp.dot(tx1, w1_ref[...], preferred_element_type=jnp.float32)
         + b_ref[...])
    h = jnp.maximum(h, 0.0)
    out_ref[...] = (jnp.dot(h, wl_ref[...], preferred_element_type=jnp.float32)
                    + bl_ref[...])


def _final(h1, y2, s, dinv, slc, w0, w1, b, wl, bl):
    return pl.pallas_call(
        _final_body,
        grid=(GRID,),
        in_specs=[
            pl.BlockSpec((BLK, H), lambda i: (i, 0)),
            pl.BlockSpec((BLK, H), lambda i: (i, 0)),
            pl.BlockSpec((NC, BLK, H), lambda i: (0, i, 0)),
            pl.BlockSpec((BLK, 1), lambda i: (i, 0)),
            pl.BlockSpec((BLK, 1), lambda i: (i, 0)),
            pl.BlockSpec((H, H), lambda i: (0, 0)),
            pl.BlockSpec((H, H), lambda i: (0, 0)),
            pl.BlockSpec((1, H), lambda i: (0, 0)),
            pl.BlockSpec((H, C), lambda i: (0, 0)),
            pl.BlockSpec((1, C), lambda i: (0, 0)),
        ],
        out_specs=[pl.BlockSpec((BLK, C), lambda i: (i, 0))],
        out_shape=[jax.ShapeDtypeStruct((NPAD, C), jnp.float32)],
    )(h1, y2, s, dinv, slc, w0, w1, b, wl, bl)[0]


def kernel(x, edge_index, W1_0, W1_1, b1, W2_0, W2_1, b2, Wl, bl):
    src2 = edge_index[0].reshape(NW, NCH, CH)
    dst2 = edge_index[1].reshape(NW, NCH, CH)
    src4 = edge_index[0].reshape(NW, NBLK, IBLK * SCH).reshape(
        NW, NBLK, IBLK, SCH)
    dst4 = edge_index[1].reshape(NW, NBLK, IBLK * SCH).reshape(
        NW, NBLK, IBLK, SCH)
    xp = jnp.concatenate(
        [x, jnp.zeros((NPAD - N, F_IN), jnp.float32)], axis=0)

    degp, slp = _deg_kernel(src2, dst2)
    degp = degp.reshape(NC, NPAD)
    slp = slp.reshape(NC, NPAD)
    dinv, slc, y1 = _prep(degp, slp, xp)
    s1 = _seg_kernel(y1, src4, dst4)
    h1, y2 = _layer(xp, y1, s1, dinv, slc, W1_0, W1_1, b1.reshape(1, H))
    s2 = _seg_kernel(y2, src4, dst4)
    out = _final(h1, y2, s2, dinv, slc, W2_0, W2_1, b2.reshape(1, H),
                 Wl, bl.reshape(1, C))
    return out[:N]


# speedup vs baseline: 1.1804x; 1.1804x over previous
import functools

import jax
import jax.numpy as jnp
from jax import lax
from jax.experimental import pallas as pl
from jax.experimental.pallas import tpu as pltpu
from jax.experimental.pallas import tpu_sc as plsc

N = 10000
E = 320000
F_IN = 128
H = 128
C = 2

NC = 2
NS = 16
NW = NC * NS
EPT = E // NW
CH = 80
NCH = EPT // CH
NPAD = 10240
STRIPE = NPAD // NS

_mesh = plsc.VectorSubcoreMesh(
    core_axis_name="c", subcore_axis_name="s", num_cores=NC, num_subcores=NS
)


@functools.partial(
    pl.kernel,
    out_type=(
        jax.ShapeDtypeStruct((NC * NPAD,), jnp.float32),
        jax.ShapeDtypeStruct((NC * NPAD,), jnp.float32),
    ),
    mesh=_mesh,
    scratch_types=[
        pltpu.VMEM((NCH, CH), jnp.int32),
        pltpu.VMEM((NCH, CH), jnp.int32),
        pltpu.VMEM((NCH, CH), jnp.float32),
        pltpu.VMEM((CH,), jnp.float32),
        pltpu.VMEM((STRIPE,), jnp.float32),
        pltpu.VMEM_SHARED((NPAD,), jnp.float32),
        pltpu.VMEM_SHARED((NPAD,), jnp.float32),
    ],
)
def _deg_kernel(src_hbm, dst_hbm, deg_out, sl_out,
                sidx, didx, slv, ones, stage, deg_s, sl_s):
    cid = lax.axis_index("c")
    sid = lax.axis_index("s")
    wid = sid * NC + cid

    one_v = jnp.ones((16,), jnp.float32)
    zero_v = jnp.zeros((16,), jnp.float32)
    for c in range(CH // 16):
        ones[pl.ds(c * 16, 16)] = one_v

    def zfill(i, _):
        stage[pl.ds(i * 16, 16)] = zero_v
        return 0
    lax.fori_loop(0, STRIPE // 16, zfill, 0)

    base = sid * STRIPE
    pltpu.sync_copy(stage, deg_s.at[pl.ds(base, STRIPE)])
    pltpu.sync_copy(stage, sl_s.at[pl.ds(base, STRIPE)])
    plsc.subcore_barrier()

    pltpu.sync_copy(src_hbm.at[wid], sidx)
    pltpu.sync_copy(dst_hbm.at[wid], didx)

    def slvals(j, _):
        for c in range(CH // 16):
            s = sidx[j, pl.ds(c * 16, 16)]
            d = didx[j, pl.ds(c * 16, 16)]
            slv[j, pl.ds(c * 16, 16)] = jnp.where(s == d, 1.0, 0.0).astype(
                jnp.float32)
        return 0
    lax.fori_loop(0, NCH, slvals, 0)

    def accum(j, _):
        pltpu.sync_copy(ones, deg_s.at[sidx.at[j]], add=True)
        pltpu.sync_copy(slv.at[j], sl_s.at[sidx.at[j]], add=True)
        return 0
    lax.fori_loop(0, NCH, accum, 0)
    plsc.subcore_barrier()

    pltpu.sync_copy(deg_s.at[pl.ds(base, STRIPE)], stage)
    pltpu.sync_copy(stage, deg_out.at[pl.ds(cid * NPAD + base, STRIPE)])
    pltpu.sync_copy(sl_s.at[pl.ds(base, STRIPE)], stage)
    pltpu.sync_copy(stage, sl_out.at[pl.ds(cid * NPAD + base, STRIPE)])


SCH = 80
IBLK = 25
NBLK = EPT // (IBLK * SCH)
ZCH = 80


@functools.partial(
    pl.kernel,
    out_type=jax.ShapeDtypeStruct((NC, NPAD, H), jnp.float32),
    mesh=_mesh,
    scratch_types=[
        pltpu.VMEM((IBLK, SCH), jnp.int32),
        pltpu.VMEM((IBLK, SCH), jnp.int32),
        pltpu.VMEM((SCH, H), jnp.float32),
        pltpu.VMEM((SCH, H), jnp.float32),
        pltpu.VMEM_SHARED((NPAD, H), jnp.float32),
        pltpu.SemaphoreType.DMA,
        pltpu.SemaphoreType.DMA,
        pltpu.SemaphoreType.DMA,
        pltpu.SemaphoreType.DMA,
    ],
)
def _seg_kernel(y_hbm, src_hbm, dst_hbm, s_out, sidx, didx, rows0, rows1, S,
                sem0, sem1, ssem0, ssem1):
    cid = lax.axis_index("c")
    sid = lax.axis_index("s")
    wid = sid * NC + cid

    zero_v = jnp.zeros((16,), jnp.float32)

    def zrow(r, _):
        for c in range(H // 16):
            rows0[r, pl.ds(c * 16, 16)] = zero_v
        return 0
    lax.fori_loop(0, ZCH, zrow, 0)

    base = sid * STRIPE
    zsrc = rows0.at[pl.ds(0, ZCH), :]

    def zstripe(i, _):
        pltpu.sync_copy(zsrc, S.at[pl.ds(base + i * ZCH, ZCH), :])
        return 0
    lax.fori_loop(0, STRIPE // ZCH, zstripe, 0)
    plsc.subcore_barrier()

    def blk(b, _):
        pltpu.sync_copy(src_hbm.at[wid, b], sidx)
        pltpu.sync_copy(dst_hbm.at[wid, b], didx)
        pltpu.async_copy(y_hbm.at[sidx.at[0]], rows0, sem0)
        pltpu.async_copy(y_hbm.at[sidx.at[1]], rows1, sem1)

        def pair(i, _):
            c0 = 2 * i
            c1 = c0 + 1
            pltpu.make_async_copy(y_hbm.at[sidx.at[c0]], rows0, sem0).wait()
            pltpu.sync_copy(rows0, S.at[didx.at[c0]], add=True)
            pltpu.async_copy(y_hbm.at[sidx.at[c0 + 2]], rows0, sem0)
            pltpu.make_async_copy(y_hbm.at[sidx.at[c1]], rows1, sem1).wait()
            pltpu.sync_copy(rows1, S.at[didx.at[c1]], add=True)

            @pl.when(c1 + 2 < IBLK)
            def _():
                pltpu.async_copy(y_hbm.at[sidx.at[c1 + 2]], rows1, sem1)
            return 0
        lax.fori_loop(0, IBLK // 2, pair, 0)
        last = IBLK - 1
        pltpu.make_async_copy(y_hbm.at[sidx.at[last]], rows0, sem0).wait()
        pltpu.sync_copy(rows0, S.at[didx.at[last]], add=True)
        return 0
    lax.fori_loop(0, NBLK, blk, 0)
    plsc.subcore_barrier()

    def rd(i, _):
        pltpu.sync_copy(S.at[pl.ds(base + i * ZCH, ZCH), :],
                        s_out.at[cid, pl.ds(base + i * ZCH, ZCH), :])
        return 0
    lax.fori_loop(0, STRIPE // ZCH, rd, 0)


BLK = 1280
GRID = NPAD // BLK


def _prep_body(degp_ref, slp_ref, x_ref, dinv_ref, slc_ref, y_ref):
    deg_full = degp_ref[0, :] + degp_ref[1, :]
    slc = slp_ref[0, :] + slp_ref[1, :]
    deg = deg_full - slc
    dinv = jnp.where(deg > 0, lax.rsqrt(jnp.maximum(deg, 1e-30)), 0.0)
    dinv_ref[...] = dinv[:, None]
    slc_ref[...] = slc[:, None]
    y_ref[...] = dinv[:, None] * x_ref[...]


def _prep(degp, slp, xp):
    return pl.pallas_call(
        _prep_body,
        grid=(GRID,),
        in_specs=[
            pl.BlockSpec((NC, BLK), lambda i: (0, i)),
            pl.BlockSpec((NC, BLK), lambda i: (0, i)),
            pl.BlockSpec((BLK, H), lambda i: (i, 0)),
        ],
        out_specs=[
            pl.BlockSpec((BLK, 1), lambda i: (i, 0)),
            pl.BlockSpec((BLK, 1), lambda i: (i, 0)),
            pl.BlockSpec((BLK, H), lambda i: (i, 0)),
        ],
        out_shape=[
            jax.ShapeDtypeStruct((NPAD, 1), jnp.float32),
            jax.ShapeDtypeStruct((NPAD, 1), jnp.float32),
            jax.ShapeDtypeStruct((NPAD, H), jnp.float32),
        ],
    )(degp, slp, xp)


def _layer_body(x_ref, y_ref, s_ref, dinv_ref, slc_ref, w0_ref, w1_ref, b_ref,
                h_ref, y2_ref):
    dinv = dinv_ref[...]
    tx1 = -dinv * (s_ref[0] + s_ref[1] - slc_ref[...] * y_ref[...])
    h = (jnp.dot(x_ref[...], w0_ref[...], preferred_element_type=jnp.float32)
         + jnp.dot(tx1, w1_ref[...], preferred_element_type=jnp.float32)
         + b_ref[...])
    h = jnp.maximum(h, 0.0)
    h_ref[...] = h
    y2_ref[...] = dinv * h


def _layer(xp, y, s, dinv, slc, w0, w1, b):
    return pl.pallas_call(
        _layer_body,
        grid=(GRID,),
        in_specs=[
            pl.BlockSpec((BLK, H), lambda i: (i, 0)),
            pl.BlockSpec((BLK, H), lambda i: (i, 0)),
            pl.BlockSpec((NC, BLK, H), lambda i: (0, i, 0)),
            pl.BlockSpec((BLK, 1), lambda i: (i, 0)),
            pl.BlockSpec((BLK, 1), lambda i: (i, 0)),
            pl.BlockSpec((H, H), lambda i: (0, 0)),
            pl.BlockSpec((H, H), lambda i: (0, 0)),
            pl.BlockSpec((1, H), lambda i: (0, 0)),
        ],
        out_specs=[
            pl.BlockSpec((BLK, H), lambda i: (i, 0)),
            pl.BlockSpec((BLK, H), lambda i: (i, 0)),
        ],
        out_shape=[
            jax.ShapeDtypeStruct((NPAD, H), jnp.float32),
            jax.ShapeDtypeStruct((NPAD, H), jnp.float32),
        ],
    )(xp, y, s, dinv, slc, w0, w1, b)


def _final_body(h_ref, y_ref, s_ref, dinv_ref, slc_ref, w0_ref, w1_ref, b_ref,
                wl_ref, bl_ref, out_ref):
    dinv = dinv_ref[...]
    tx1 = -dinv * (s_ref[0] + s_ref[1] - slc_ref[...] * y_ref[...])
    h = (jnp.dot(h_ref[...], w0_ref[...], preferred_element_type=jnp.float32)
         + jnp.dot(tx1, w1_ref[...], preferred_element_type=jnp.float32)
         + b_ref[...])
    h = jnp.maximum(h, 0.0)
    out_ref[...] = (jnp.dot(h, wl_ref[...], preferred_element_type=jnp.float32)
                    + bl_ref[...])


def _final(h1, y2, s, dinv, slc, w0, w1, b, wl, bl):
    return pl.pallas_call(
        _final_body,
        grid=(GRID,),
        in_specs=[
            pl.BlockSpec((BLK, H), lambda i: (i, 0)),
            pl.BlockSpec((BLK, H), lambda i: (i, 0)),
            pl.BlockSpec((NC, BLK, H), lambda i: (0, i, 0)),
            pl.BlockSpec((BLK, 1), lambda i: (i, 0)),
            pl.BlockSpec((BLK, 1), lambda i: (i, 0)),
            pl.BlockSpec((H, H), lambda i: (0, 0)),
            pl.BlockSpec((H, H), lambda i: (0, 0)),
            pl.BlockSpec((1, H), lambda i: (0, 0)),
            pl.BlockSpec((H, C), lambda i: (0, 0)),
            pl.BlockSpec((1, C), lambda i: (0, 0)),
        ],
        out_specs=[pl.BlockSpec((BLK, C), lambda i: (i, 0))],
        out_shape=[jax.ShapeDtypeStruct((NPAD, C), jnp.float32)],
    )(h1, y2, s, dinv, slc, w0, w1, b, wl, bl)[0]


def kernel(x, edge_index, W1_0, W1_1, b1, W2_0, W2_1, b2, Wl, bl):
    src2 = edge_index[0].reshape(NW, NCH, CH)
    dst2 = edge_index[1].reshape(NW, NCH, CH)
    src4 = edge_index[0].reshape(NW, NBLK, IBLK * SCH).reshape(
        NW, NBLK, IBLK, SCH)
    dst4 = edge_index[1].reshape(NW, NBLK, IBLK * SCH).reshape(
        NW, NBLK, IBLK, SCH)
    xp = jnp.concatenate(
        [x, jnp.zeros((NPAD - N, F_IN), jnp.float32)], axis=0)

    degp, slp = _deg_kernel(src2, dst2)
    degp = degp.reshape(NC, NPAD)
    slp = slp.reshape(NC, NPAD)
    dinv, slc, y1 = _prep(degp, slp, xp)
    s1 = _seg_kernel(y1, src4, dst4)
    h1, y2 = _layer(xp, y1, s1, dinv, slc, W1_0, W1_1, b1.reshape(1, H))
    s2 = _seg_kernel(y2, src4, dst4)
    out = _final(h1, y2, s2, dinv, slc, W2_0, W2_1, b2.reshape(1, H),
                 Wl, bl.reshape(1, C))
    return out[:N]
